# speedup vs baseline: 2.3573x; 2.3573x over previous
import jax
import jax.numpy as jnp
from jax.experimental import pallas as pl
from jax.experimental.pallas import tpu as pltpu

D_MODEL = 1024
N_EXP = 8
TOKEN_BLOCK = 256


def _moe_dense_body(x_ref, gw_ref, gb_ref, w_ref, b_ref, o_ref):
    x = x_ref[...]
    logits = jax.lax.dot_general(
        x, gw_ref[...], (((1,), (1,)), ((), ())),
        preferred_element_type=jnp.float32) + gb_ref[...]
    iota = jax.lax.broadcasted_iota(jnp.int32, logits.shape, 1)
    m1 = jnp.max(logits, axis=1, keepdims=True)
    i1 = jnp.min(jnp.where(logits == m1, iota, N_EXP), axis=1, keepdims=True)
    l2 = jnp.where(iota == i1, -1e30, logits)
    m2 = jnp.max(l2, axis=1, keepdims=True)
    i2 = jnp.min(jnp.where(l2 == m2, iota, N_EXP), axis=1, keepdims=True)
    e2 = jnp.exp(m2 - m1)
    w1 = 1.0 / (1.0 + e2)
    w2 = e2 / (1.0 + e2)

    acc = jnp.zeros_like(o_ref)
    for e in range(N_EXP):
        y = jax.lax.dot_general(
            x, w_ref[e], (((1,), (1,)), ((), ())),
            preferred_element_type=jnp.float32) + b_ref[e:e + 1, :]
        coeff = (w1 * (i1 == e).astype(jnp.float32)
                 + w2 * (i2 == e).astype(jnp.float32))
        acc = acc + coeff * y
    o_ref[...] = acc


def kernel(x, gate_w, gate_b, expert_w, expert_b, *, interpret=False):
    batch, seq, d = x.shape
    xf = x.reshape(-1, d)
    n = xf.shape[0]
    out = pl.pallas_call(
        _moe_dense_body,
        grid=(n // TOKEN_BLOCK,),
        in_specs=[
            pl.BlockSpec((TOKEN_BLOCK, d), lambda i: (i, 0)),
            pl.BlockSpec((N_EXP, d), lambda i: (0, 0)),
            pl.BlockSpec((1, N_EXP), lambda i: (0, 0)),
            pl.BlockSpec((N_EXP, d, d), lambda i: (0, 0, 0)),
            pl.BlockSpec((N_EXP, d), lambda i: (0, 0)),
        ],
        out_specs=pl.BlockSpec((TOKEN_BLOCK, d), lambda i: (i, 0)),
        out_shape=jax.ShapeDtypeStruct((n, d), jnp.float32),
        interpret=interpret,
    )(xf, gate_w, gate_b.reshape(1, N_EXP), expert_w, expert_b)
    return out.reshape(batch, seq, d)
